# Initial kernel scaffold; baseline (speedup 1.0000x reference)
import jax
import jax.numpy as jnp
from jax.experimental import pallas as pl


def kernel(input, edge_index, edge_vals, W, b):
    raise NotImplementedError("write your pallas kernel here")


# speedup vs baseline: 4.4558x; 4.4558x over previous
import jax
import jax.numpy as jnp
from jax import lax
from jax.experimental import pallas as pl
from jax.experimental.pallas import tpu as pltpu
from jax.experimental.pallas import tpu_sc as plsc

N = 10000
E = 320000
D = 128
NC = 2
NS = 16
NW = NC * NS
CHUNK = 80
EPT = E // NW
NCHUNK = EPT // CHUNK
NROWCHUNK = N // CHUNK
ROUNDS = -(-NROWCHUNK // NS)


def _matmul_body(x_ref, w_ref, o_ref):
    o_ref[...] = jnp.dot(x_ref[...], w_ref[...],
                         preferred_element_type=jnp.float32)


_matmul = pl.pallas_call(
    _matmul_body,
    grid=(10,),
    in_specs=[pl.BlockSpec((N // 10, D), lambda i: (i, 0)),
              pl.BlockSpec((D, D), lambda i: (0, 0))],
    out_specs=pl.BlockSpec((N // 10, D), lambda i: (i, 0)),
    out_shape=jax.ShapeDtypeStruct((N, D), jnp.float32),
)


def _combine_body(p0_ref, p1_ref, b_ref, o_ref):
    o_ref[...] = p0_ref[0] + p1_ref[0] + b_ref[...]


_combine = pl.pallas_call(
    _combine_body,
    grid=(10,),
    in_specs=[pl.BlockSpec((1, N // 10, D), lambda i: (0, i, 0)),
              pl.BlockSpec((1, N // 10, D), lambda i: (1, i, 0)),
              pl.BlockSpec((1, D), lambda i: (0, 0))],
    out_specs=pl.BlockSpec((N // 10, D), lambda i: (i, 0)),
    out_shape=jax.ShapeDtypeStruct((N, D), jnp.float32),
)


def _sc_body(hid_hbm, src_hbm, dst_hbm, val_hbm, out_hbm,
             src_v, dst_v, val_v, rows_v, accum_sh, sem):
    c = lax.axis_index("c")
    s = lax.axis_index("s")
    w = c * NS + s

    def zrow(r, carry):
        for k in range(D // 16):
            rows_v[r, pl.ds(k * 16, 16)] = jnp.zeros((16,), jnp.float32)
        return carry
    lax.fori_loop(0, CHUNK, zrow, 0)
    for t in range(ROUNDS):
        idx = t * NS + s
        @pl.when(idx < NROWCHUNK)
        def _():
            pltpu.sync_copy(rows_v, accum_sh.at[pl.ds(idx * CHUNK, CHUNK)])
    plsc.subcore_barrier()

    def chunk(j, carry):
        base = w * EPT + j * CHUNK
        pltpu.sync_copy(src_hbm.at[pl.ds(base, CHUNK)], src_v)
        pltpu.sync_copy(dst_hbm.at[pl.ds(base, CHUNK)], dst_v)
        pltpu.sync_copy(val_hbm.at[pl.ds(base, CHUNK)], val_v)
        pltpu.async_copy(hid_hbm.at[src_v], rows_v, sem).wait()

        def scale(g, inner):
            vv = val_v[pl.ds(g * 16, 16)]
            for i in range(16):
                v = vv[i]
                r = g * 16 + i
                for k in range(D // 16):
                    rows_v[r, pl.ds(k * 16, 16)] = (
                        rows_v[r, pl.ds(k * 16, 16)] * v)
            return inner
        lax.fori_loop(0, CHUNK // 16, scale, 0)

        pltpu.sync_copy(rows_v, accum_sh.at[dst_v], add=True)
        return carry
    lax.fori_loop(0, NCHUNK, chunk, 0)

    plsc.subcore_barrier()
    for t in range(ROUNDS):
        idx = t * NS + s
        @pl.when(idx < NROWCHUNK)
        def _():
            pltpu.sync_copy(accum_sh.at[pl.ds(idx * CHUNK, CHUNK)],
                            out_hbm.at[c, pl.ds(idx * CHUNK, CHUNK)])


_sc_call = pl.kernel(
    _sc_body,
    out_type=jax.ShapeDtypeStruct((NC, N, D), jnp.float32),
    mesh=plsc.VectorSubcoreMesh(core_axis_name="c", subcore_axis_name="s"),
    scratch_types=[
        pltpu.VMEM((CHUNK,), jnp.int32),
        pltpu.VMEM((CHUNK,), jnp.int32),
        pltpu.VMEM((CHUNK,), jnp.float32),
        pltpu.VMEM((CHUNK, D), jnp.float32),
        pltpu.VMEM_SHARED((N, D), jnp.float32),
        pltpu.SemaphoreType.DMA,
    ],
)


def kernel(input, edge_index, edge_vals, W, b):
    ei = edge_index.astype(jnp.int32)
    dst = ei[0]
    src = ei[1]
    hidden = _matmul(input, W)
    partials = _sc_call(hidden, src, dst, edge_vals)
    return _combine(partials, partials, b)


# speedup vs baseline: 9.8389x; 2.2081x over previous
import jax
import jax.numpy as jnp
from jax import lax
from jax.experimental import pallas as pl
from jax.experimental.pallas import tpu as pltpu
from jax.experimental.pallas import tpu_sc as plsc

N = 10000
E = 320000
D = 128
NC = 2
NS = 16
NW = NC * NS
CHUNK = 80
EPT = E // NW
NCHUNK = EPT // CHUNK
NROWCHUNK = N // CHUNK
ROUNDS = -(-NROWCHUNK // NS)
SUP = 25
NSUP = NCHUNK // SUP


def _matmul_body(x_ref, w_ref, o_ref):
    o_ref[...] = jnp.dot(x_ref[...], w_ref[...],
                         preferred_element_type=jnp.float32)


_matmul = pl.pallas_call(
    _matmul_body,
    grid=(10,),
    in_specs=[pl.BlockSpec((N // 10, D), lambda i: (i, 0)),
              pl.BlockSpec((D, D), lambda i: (0, 0))],
    out_specs=pl.BlockSpec((N // 10, D), lambda i: (i, 0)),
    out_shape=jax.ShapeDtypeStruct((N, D), jnp.float32),
)


def _combine_body(p0_ref, p1_ref, b_ref, o_ref):
    o_ref[...] = p0_ref[0] + p1_ref[0] + b_ref[...]


_combine = pl.pallas_call(
    _combine_body,
    grid=(10,),
    in_specs=[pl.BlockSpec((1, N // 10, D), lambda i: (0, i, 0)),
              pl.BlockSpec((1, N // 10, D), lambda i: (1, i, 0)),
              pl.BlockSpec((1, D), lambda i: (0, 0))],
    out_specs=pl.BlockSpec((N // 10, D), lambda i: (i, 0)),
    out_shape=jax.ShapeDtypeStruct((N, D), jnp.float32),
)


def _sc_body(hid_hbm, src_hbm, dst_hbm, val_hbm, out_hbm,
             src_v, dst_v, val_v, rows0, rows1, accum_sh, sem0, sem1):
    c = lax.axis_index("c")
    s = lax.axis_index("s")
    w = c * NS + s

    def zrow(r, carry):
        for k in range(D // 16):
            rows0[r, pl.ds(k * 16, 16)] = jnp.zeros((16,), jnp.float32)
        return carry
    lax.fori_loop(0, CHUNK, zrow, 0)
    for t in range(ROUNDS):
        idx = t * NS + s
        @pl.when(idx < NROWCHUNK)
        def _():
            pltpu.sync_copy(rows0, accum_sh.at[pl.ds(idx * CHUNK, CHUNK)])
    plsc.subcore_barrier()

    bufs = ((rows0, sem0), (rows1, sem1))

    def process(j, rows, sem, nrows, nsem):
        @pl.when(j + 1 < SUP)
        def _():
            pltpu.async_copy(hid_hbm.at[src_v.at[j + 1]], nrows, nsem)
        pltpu.make_async_copy(hid_hbm.at[src_v.at[j]], rows, sem).wait()

        def scale(g, inner):
            vv = val_v[j, pl.ds(g * 16, 16)]
            for i in range(16):
                v = vv[i]
                r = g * 16 + i
                for k in range(D // 16):
                    rows[r, pl.ds(k * 16, 16)] = rows[r, pl.ds(k * 16, 16)] * v
            return inner
        lax.fori_loop(0, CHUNK // 16, scale, 0)

        pltpu.sync_copy(rows, accum_sh.at[dst_v.at[j]], add=True)

    def superchunk(ss, carry):
        pltpu.sync_copy(src_hbm.at[w, ss], src_v)
        pltpu.sync_copy(dst_hbm.at[w, ss], dst_v)
        pltpu.sync_copy(val_hbm.at[w, ss], val_v)
        pltpu.async_copy(hid_hbm.at[src_v.at[0]], rows0, sem0)

        def outer(g2, c2):
            for b in range(2):
                j = g2 * 2 + b
                rows, sem = bufs[b]
                nrows, nsem = bufs[1 - b]
                @pl.when(j < SUP)
                def _():
                    process(j, rows, sem, nrows, nsem)
            return c2
        lax.fori_loop(0, (SUP + 1) // 2, outer, 0)
        return carry
    lax.fori_loop(0, NSUP, superchunk, 0)

    plsc.subcore_barrier()
    for t in range(ROUNDS):
        idx = t * NS + s
        @pl.when(idx < NROWCHUNK)
        def _():
            pltpu.sync_copy(accum_sh.at[pl.ds(idx * CHUNK, CHUNK)],
                            out_hbm.at[c, pl.ds(idx * CHUNK, CHUNK)])


_sc_call = pl.kernel(
    _sc_body,
    out_type=jax.ShapeDtypeStruct((NC, N, D), jnp.float32),
    mesh=plsc.VectorSubcoreMesh(core_axis_name="c", subcore_axis_name="s"),
    scratch_types=[
        pltpu.VMEM((SUP, CHUNK), jnp.int32),
        pltpu.VMEM((SUP, CHUNK), jnp.int32),
        pltpu.VMEM((SUP, CHUNK), jnp.float32),
        pltpu.VMEM((CHUNK, D), jnp.float32),
        pltpu.VMEM((CHUNK, D), jnp.float32),
        pltpu.VMEM_SHARED((N, D), jnp.float32),
        pltpu.SemaphoreType.DMA,
        pltpu.SemaphoreType.DMA,
    ],
)


def kernel(input, edge_index, edge_vals, W, b):
    ei = edge_index.astype(jnp.int32)
    dst3 = ei[0].reshape(NW, NSUP, SUP, CHUNK)
    src3 = ei[1].reshape(NW, NSUP, SUP, CHUNK)
    val3 = edge_vals.reshape(NW, NSUP, SUP, CHUNK)
    hidden = _matmul(input, W)
    partials = _sc_call(hidden, src3, dst3, val3)
    return _combine(partials, partials, b)


# speedup vs baseline: 10.5226x; 1.0695x over previous
import jax
import jax.numpy as jnp
from jax import lax
from jax.experimental import pallas as pl
from jax.experimental.pallas import tpu as pltpu
from jax.experimental.pallas import tpu_sc as plsc

N = 10000
E = 320000
D = 128
NC = 2
NS = 16
NW = NC * NS
CHUNK = 80
EPT = E // NW
NCHUNK = EPT // CHUNK
NROWCHUNK = N // CHUNK
ROUNDS = -(-NROWCHUNK // NS)
SUP = 25
NSUP = NCHUNK // SUP


def _matmul_body(x_ref, w_ref, o_ref):
    o_ref[...] = jnp.dot(x_ref[...], w_ref[...],
                         preferred_element_type=jnp.float32)


_matmul = pl.pallas_call(
    _matmul_body,
    grid=(10,),
    in_specs=[pl.BlockSpec((N // 10, D), lambda i: (i, 0)),
              pl.BlockSpec((D, D), lambda i: (0, 0))],
    out_specs=pl.BlockSpec((N // 10, D), lambda i: (i, 0)),
    out_shape=jax.ShapeDtypeStruct((N, D), jnp.float32),
)


def _combine_body(p0_ref, p1_ref, b_ref, o_ref):
    o_ref[...] = p0_ref[0] + p1_ref[0] + b_ref[...]


_combine = pl.pallas_call(
    _combine_body,
    grid=(10,),
    in_specs=[pl.BlockSpec((1, N // 10, D), lambda i: (0, i, 0)),
              pl.BlockSpec((1, N // 10, D), lambda i: (1, i, 0)),
              pl.BlockSpec((1, D), lambda i: (0, 0))],
    out_specs=pl.BlockSpec((N // 10, D), lambda i: (i, 0)),
    out_shape=jax.ShapeDtypeStruct((N, D), jnp.float32),
)


def _sc_body(hid_hbm, src_hbm, dst_hbm, val_hbm, out_hbm,
             src_v, dst_v, val_v, rows0, rows1, rows2, accum_sh,
             gsem0, gsem1, gsem2, ssem0, ssem1, ssem2):
    c = lax.axis_index("c")
    s = lax.axis_index("s")
    w = c * NS + s
    rows = (rows0, rows1, rows2)
    gsem = (gsem0, gsem1, gsem2)
    ssem = (ssem0, ssem1, ssem2)

    def zrow(r, carry):
        for k in range(D // 16):
            rows0[r, pl.ds(k * 16, 16)] = jnp.zeros((16,), jnp.float32)
        return carry
    lax.fori_loop(0, CHUNK, zrow, 0)
    for t in range(ROUNDS):
        idx = t * NS + s
        @pl.when(idx < NROWCHUNK)
        def _():
            pltpu.sync_copy(rows0, accum_sh.at[pl.ds(idx * CHUNK, CHUNK)])
    plsc.subcore_barrier()

    def process(j, b):
        pb = (b + 2) % 3
        @pl.when(j >= 1)
        def _():
            pltpu.make_async_copy(
                rows[pb], accum_sh.at[dst_v.at[j - 1]], ssem[pb]).wait()
        @pl.when(j + 2 < SUP)
        def _():
            pltpu.async_copy(hid_hbm.at[src_v.at[j + 2]], rows[pb], gsem[pb])
        pltpu.make_async_copy(hid_hbm.at[src_v.at[j]], rows[b], gsem[b]).wait()

        def scale(g, inner):
            vv = val_v[j, pl.ds(g * 16, 16)]
            for i in range(16):
                v = vv[i]
                r = g * 16 + i
                for k in range(D // 16):
                    rows[b][r, pl.ds(k * 16, 16)] = (
                        rows[b][r, pl.ds(k * 16, 16)] * v)
            return inner
        lax.fori_loop(0, CHUNK // 16, scale, 0)

        pltpu.async_copy(rows[b], accum_sh.at[dst_v.at[j]], ssem[b])

    def superchunk(ss, carry):
        pltpu.sync_copy(src_hbm.at[w, ss], src_v)
        pltpu.sync_copy(dst_hbm.at[w, ss], dst_v)
        pltpu.sync_copy(val_hbm.at[w, ss], val_v)
        pltpu.async_copy(hid_hbm.at[src_v.at[0]], rows0, gsem0)
        pltpu.async_copy(hid_hbm.at[src_v.at[1]], rows1, gsem1)

        def outer(g3, c2):
            for b in range(3):
                j = g3 * 3 + b
                @pl.when(j < SUP)
                def _():
                    process(j, b)
            return c2
        lax.fori_loop(0, -(-SUP // 3), outer, 0)
        pltpu.make_async_copy(
            rows[(SUP - 1) % 3], accum_sh.at[dst_v.at[SUP - 1]],
            ssem[(SUP - 1) % 3]).wait()
        return carry
    lax.fori_loop(0, NSUP, superchunk, 0)

    plsc.subcore_barrier()
    for t in range(ROUNDS):
        idx = t * NS + s
        @pl.when(idx < NROWCHUNK)
        def _():
            pltpu.sync_copy(accum_sh.at[pl.ds(idx * CHUNK, CHUNK)],
                            out_hbm.at[c, pl.ds(idx * CHUNK, CHUNK)])


_sc_call = pl.kernel(
    _sc_body,
    out_type=jax.ShapeDtypeStruct((NC, N, D), jnp.float32),
    mesh=plsc.VectorSubcoreMesh(core_axis_name="c", subcore_axis_name="s"),
    scratch_types=[
        pltpu.VMEM((SUP, CHUNK), jnp.int32),
        pltpu.VMEM((SUP, CHUNK), jnp.int32),
        pltpu.VMEM((SUP, CHUNK), jnp.float32),
        pltpu.VMEM((CHUNK, D), jnp.float32),
        pltpu.VMEM((CHUNK, D), jnp.float32),
        pltpu.VMEM((CHUNK, D), jnp.float32),
        pltpu.VMEM_SHARED((N, D), jnp.float32),
        pltpu.SemaphoreType.DMA,
        pltpu.SemaphoreType.DMA,
        pltpu.SemaphoreType.DMA,
        pltpu.SemaphoreType.DMA,
        pltpu.SemaphoreType.DMA,
        pltpu.SemaphoreType.DMA,
    ],
)


def kernel(input, edge_index, edge_vals, W, b):
    ei = edge_index.astype(jnp.int32)
    dst3 = ei[0].reshape(NW, NSUP, SUP, CHUNK)
    src3 = ei[1].reshape(NW, NSUP, SUP, CHUNK)
    val3 = edge_vals.reshape(NW, NSUP, SUP, CHUNK)
    hidden = _matmul(input, W)
    partials = _sc_call(hidden, src3, dst3, val3)
    return _combine(partials, partials, b)
